# baseline (speedup 1.0000x reference)
import functools

import jax
import jax.numpy as jnp
from jax import lax
from jax.experimental import pallas as pl
from jax.experimental.pallas import tpu as pltpu
from jax.experimental.pallas import tpu_sc as plsc

_NC = 2
_NS = 16
_L = 16
_NW = _NC * _NS

_B = 16384
_D = 64
_BPW = _B // _NW
_CH = 128
_NCH = _BPW // _CH


def _neg_sqrt(x):
    i = lax.bitcast_convert_type(x, jnp.int32)
    i = jnp.int32(0x5F3759DF) - lax.shift_right_logical(i, 1)
    y = lax.bitcast_convert_type(i, jnp.float32)
    for _ in range(3):
        y = y * (1.5 - 0.5 * x * y * y)
    return -(x * y)


def _body(users_h, pos_h, neg_h, rel_h, ent_h, relemb_h, outp_h, outn_h,
          uidx, pidx, nidx, ridx, u_r, r_r, p_r, n_r, op_v, on_v, sem):
    wid = lax.axis_index("s") * _NC + lax.axis_index("c")
    base = wid * _BPW
    pltpu.sync_copy(users_h.at[pl.ds(base, _BPW)], uidx)
    pltpu.sync_copy(pos_h.at[pl.ds(base, _BPW)], pidx)
    pltpu.sync_copy(neg_h.at[pl.ds(base, _BPW)], nidx)
    pltpu.sync_copy(rel_h.at[pl.ds(base, _BPW)], ridx)

    lane = lax.iota(jnp.int32, _L)
    zero = jnp.zeros((_L,), jnp.float32)

    for c in range(_NCH):
        off = c * _CH
        cp_u = pltpu.async_copy(ent_h.at[uidx.at[pl.ds(off, _CH)]], u_r, sem)
        cp_r = pltpu.async_copy(relemb_h.at[ridx.at[pl.ds(off, _CH)]], r_r, sem)
        cp_p = pltpu.async_copy(ent_h.at[pidx.at[pl.ds(off, _CH)]], p_r, sem)
        cp_n = pltpu.async_copy(ent_h.at[nidx.at[pl.ds(off, _CH)]], n_r, sem)
        cp_u.wait()
        cp_r.wait()
        cp_p.wait()
        cp_n.wait()

        def group(g, _):
            row = g * _L + lane

            def dstep(dd, carry):
                ap, an = carry
                col = jnp.full((_L,), dd, jnp.int32)
                uv = plsc.load_gather(u_r, [row, col])
                rv = plsc.load_gather(r_r, [row, col])
                pv = plsc.load_gather(p_r, [row, col])
                nv = plsc.load_gather(n_r, [row, col])
                t = uv + rv
                dp = t - pv
                dn = t - nv
                return ap + dp * dp, an + dn * dn

            ap, an = lax.fori_loop(0, _D, dstep, (zero, zero))
            op_v[pl.ds(off + g * _L, _L)] = _neg_sqrt(ap)
            on_v[pl.ds(off + g * _L, _L)] = _neg_sqrt(an)
            return 0

        lax.fori_loop(0, _CH // _L, group, 0)

    pltpu.sync_copy(op_v, outp_h.at[pl.ds(base, _BPW)])
    pltpu.sync_copy(on_v, outn_h.at[pl.ds(base, _BPW)])


_sc_score = functools.partial(
    pl.kernel,
    out_type=(jax.ShapeDtypeStruct((_B,), jnp.float32),
              jax.ShapeDtypeStruct((_B,), jnp.float32)),
    mesh=plsc.VectorSubcoreMesh(core_axis_name="c", subcore_axis_name="s"),
    compiler_params=pltpu.CompilerParams(needs_layout_passes=False,
                                         use_tc_tiling_on_sc=False),
    scratch_types=[
        pltpu.VMEM((_BPW,), jnp.int32),
        pltpu.VMEM((_BPW,), jnp.int32),
        pltpu.VMEM((_BPW,), jnp.int32),
        pltpu.VMEM((_BPW,), jnp.int32),
        pltpu.VMEM((_CH, _D), jnp.float32),
        pltpu.VMEM((_CH, _D), jnp.float32),
        pltpu.VMEM((_CH, _D), jnp.float32),
        pltpu.VMEM((_CH, _D), jnp.float32),
        pltpu.VMEM((_BPW,), jnp.float32),
        pltpu.VMEM((_BPW,), jnp.float32),
        pltpu.SemaphoreType.DMA,
    ],
)(_body)


def kernel(users, pos_items, neg_items, relations, ent_emb, rel_emb):
    return _sc_score(users.astype(jnp.int32), pos_items.astype(jnp.int32),
                     neg_items.astype(jnp.int32), relations.astype(jnp.int32),
                     ent_emb, rel_emb)


# speedup vs baseline: 1.6116x; 1.6116x over previous
import functools

import jax
import jax.numpy as jnp
from jax import lax
from jax.experimental import pallas as pl
from jax.experimental.pallas import tpu as pltpu
from jax.experimental.pallas import tpu_sc as plsc

_NC = 2
_NS = 16
_L = 16
_NW = _NC * _NS

_B = 16384
_D = 64
_NR = 1000
_BPW = _B // _NW
_CH = 64
_NCH = _BPW // _CH
_G = _CH // _L


def _neg_sqrt(x):
    i = lax.bitcast_convert_type(x, jnp.int32)
    i = jnp.int32(0x5F3759DF) - lax.shift_right_logical(i, 1)
    y = lax.bitcast_convert_type(i, jnp.float32)
    for _ in range(3):
        y = y * (1.5 - 0.5 * x * y * y)
    return -(x * y)


def _body(users_h, pos_h, neg_h, rel_h, ent_h, relemb_h, outp_h, outn_h,
          uidx, pidx, nidx, ridx, bufs, op_v, on_v, sem0, sem1):
    wid = lax.axis_index("s") * _NC + lax.axis_index("c")
    base = wid * _BPW
    pltpu.sync_copy(users_h.at[pl.ds(base, _BPW)], uidx)
    pltpu.sync_copy(pos_h.at[pl.ds(base, _BPW)], pidx)
    pltpu.sync_copy(neg_h.at[pl.ds(base, _BPW)], nidx)
    pltpu.sync_copy(rel_h.at[pl.ds(base, _BPW)], ridx)

    sems = (sem0, sem1)
    lane = lax.iota(jnp.int32, _L)
    zero = jnp.zeros((_L,), jnp.float32)

    def fire(c, slot):
        off = c * _CH
        sem = sems[slot]
        u_b, r_b, p_b, n_b = (bufs.at[4 * slot + j] for j in range(4))

        def grp(g, _):
            s = pl.ds(off + g * _L, _L)
            uv = uidx[s]
            rv = ridx[s]
            pv = pidx[s]
            nv = nidx[s]
            for k in range(_L):
                d = pl.ds(g * _L + k, 1)
                pltpu.async_copy(ent_h.at[pl.ds(uv[k], 1)], u_b.at[d], sem)
                pltpu.async_copy(relemb_h.at[pl.ds(rv[k], 1)], r_b.at[d], sem)
                pltpu.async_copy(ent_h.at[pl.ds(pv[k], 1)], p_b.at[d], sem)
                pltpu.async_copy(ent_h.at[pl.ds(nv[k], 1)], n_b.at[d], sem)
            return 0

        lax.fori_loop(0, _G, grp, 0)

    def drain(slot):
        sem = sems[slot]
        for j in range(4):
            pltpu.make_async_copy(ent_h.at[pl.ds(0, _CH)], bufs.at[4 * slot + j],
                                  sem).wait()

    fire(0, 0)
    for c in range(_NCH):
        slot = c % 2
        if c + 1 < _NCH:
            fire(c + 1, 1 - slot)
        drain(slot)
        off = c * _CH
        u_b, r_b, p_b, n_b = (bufs.at[4 * slot + j] for j in range(4))

        def group(g, _):
            row = g * _L + lane
            sl = pl.ds(off + g * _L, _L)

            def dblock(db, carry):
                ap, an = carry
                d0 = db * _L
                for k in range(_L):
                    col = jnp.full((_L,), d0 + k, jnp.int32)
                    uv = plsc.load_gather(u_b, [row, col])
                    rv = plsc.load_gather(r_b, [row, col])
                    pv = plsc.load_gather(p_b, [row, col])
                    nv = plsc.load_gather(n_b, [row, col])
                    t = uv + rv
                    dp = t - pv
                    dn = t - nv
                    ap = ap + dp * dp
                    an = an + dn * dn
                return ap, an

            ap, an = lax.fori_loop(0, _D // _L, dblock, (zero, zero))
            op_v[sl] = _neg_sqrt(ap)
            on_v[sl] = _neg_sqrt(an)
            return 0

        lax.fori_loop(0, _G, group, 0)

    pltpu.sync_copy(op_v, outp_h.at[pl.ds(base, _BPW)])
    pltpu.sync_copy(on_v, outn_h.at[pl.ds(base, _BPW)])


_sc_score = functools.partial(
    pl.kernel,
    out_type=(jax.ShapeDtypeStruct((_B,), jnp.float32),
              jax.ShapeDtypeStruct((_B,), jnp.float32)),
    mesh=plsc.VectorSubcoreMesh(core_axis_name="c", subcore_axis_name="s"),
    compiler_params=pltpu.CompilerParams(needs_layout_passes=False,
                                         disable_bounds_checks=True),
    scratch_types=[
        pltpu.VMEM((_BPW,), jnp.int32),
        pltpu.VMEM((_BPW,), jnp.int32),
        pltpu.VMEM((_BPW,), jnp.int32),
        pltpu.VMEM((_BPW,), jnp.int32),
        pltpu.VMEM((8, _CH, _D), jnp.float32),
        pltpu.VMEM((_BPW,), jnp.float32),
        pltpu.VMEM((_BPW,), jnp.float32),
        pltpu.SemaphoreType.DMA,
        pltpu.SemaphoreType.DMA,
    ],
)(_body)


def kernel(users, pos_items, neg_items, relations, ent_emb, rel_emb):
    return _sc_score(users.astype(jnp.int32), pos_items.astype(jnp.int32),
                     neg_items.astype(jnp.int32), relations.astype(jnp.int32),
                     ent_emb, rel_emb)
